# baseline (speedup 1.0000x reference)
import jax
import jax.numpy as jnp
from jax import lax
from jax.experimental import pallas as pl
from jax.experimental.pallas import tpu as pltpu
from jax.experimental.pallas import tpu_sc as plsc

N_NODES = 100000
D_FEAT = 128
N_GRAPHS = 1024

NC = 2
NS = 16
NW = NC * NS
LANES = 16

CHUNK = 400
HALF_A = 192
HALF_DMA = 200
NCHUNKS = N_NODES // CHUNK
NITER = (NCHUNKS + NW - 1) // NW
NGRP = CHUNK // LANES
GRP_A = HALF_A // LANES
G_PAD = N_GRAPHS + LANES


def _sc_body(x_hbm, batch_hbm, out_hbm,
             xb0, xb1, bb0, bb1, drop,
             isa0, isa1, isb0, isb1, osa0, osa1, osb0, osb1):
    wid = lax.axis_index("s") * NC + lax.axis_index("c")
    xbs, bbs = (xb0, xb1), (bb0, bb1)
    isemA, isemB = (isa0, isa1), (isb0, isb1)
    osemA, osemB = (osa0, osa1), (osb0, osb1)

    ones16 = jnp.full((LANES,), 1.0, jnp.float32)

    def init_body(g, carry):
        drop[pl.ds(g * LANES, LANES)] = ones16
        return carry

    lax.fori_loop(0, G_PAD // LANES, init_body, 0)

    def guard(it):
        return wid + it * NW < NCHUNKS

    def in_copies(it, p):
        row0 = (wid + it * NW) * CHUNK
        return (
            pltpu.make_async_copy(x_hbm.at[pl.ds(row0, HALF_DMA), :],
                                  xbs[p].at[pl.ds(0, HALF_DMA), :], isemA[p]),
            pltpu.make_async_copy(batch_hbm.at[pl.ds(row0, CHUNK)], bbs[p], isemA[p]),
            pltpu.make_async_copy(x_hbm.at[pl.ds(row0 + HALF_DMA, HALF_DMA), :],
                                  xbs[p].at[pl.ds(HALF_DMA, HALF_DMA), :], isemB[p]),
        )

    def out_copies(it, p):
        row0 = (wid + it * NW) * CHUNK
        return (
            pltpu.make_async_copy(xbs[p].at[pl.ds(0, HALF_A), :],
                                  out_hbm.at[pl.ds(row0, HALF_A), :], osemA[p]),
            pltpu.make_async_copy(xbs[p].at[pl.ds(HALF_A, CHUNK - HALF_A), :],
                                  out_hbm.at[pl.ds(row0 + HALF_A, CHUNK - HALF_A), :],
                                  osemB[p]),
        )

    def start_in(it, p):
        @pl.when(guard(it))
        def _():
            ca, cb, cc = in_copies(it, p)
            ca.start()
            cb.start()
            cc.start()

    def wait_out(it, p):
        @pl.when(guard(it))
        def _():
            oa, ob = out_copies(it, p)
            oa.wait()
            ob.wait()

    def load_group(bbuf, g):
        bvec = bbuf[pl.ds(g * LANES, LANES)]
        b_first = bvec[0]
        b_last = bvec[LANES - 1]
        dval = drop[pl.ds(b_first, LANES)][0]
        return bvec, b_first == b_last, dval

    def compute_groups(xbuf, bbuf, g_lo, g_hi):
        def grp_body(g, carry):
            bvec, same, dval = carry
            gn = jnp.minimum(g + 1, g_hi - 1)
            carry_next = load_group(bbuf, gn)

            @pl.when(same)
            def _():
                for j in range(LANES):
                    r = g * LANES + j
                    for k in range(D_FEAT // LANES):
                        xv = xbuf[r, pl.ds(k * LANES, LANES)]
                        xbuf[r, pl.ds(k * LANES, LANES)] = xv * dval

            @pl.when(jnp.logical_not(same))
            def _():
                for j in range(LANES):
                    b = bvec[j]
                    dv = drop[pl.ds(b, LANES)][0]
                    r = g * LANES + j
                    for k in range(D_FEAT // LANES):
                        xv = xbuf[r, pl.ds(k * LANES, LANES)]
                        xbuf[r, pl.ds(k * LANES, LANES)] = xv * dv
            return carry_next

        lax.fori_loop(g_lo, g_hi, grp_body, load_group(bbuf, g_lo))

    def process(it, p):
        @pl.when(guard(it))
        def _():
            ca, cb, _cc = in_copies(it, p)
            oa, ob = out_copies(it, p)
            ca.wait()
            cb.wait()
            compute_groups(xbs[p], bbs[p], 0, GRP_A)
            oa.start()
            _ca, _cb, cc = in_copies(it, p)
            cc.wait()
            compute_groups(xbs[p], bbs[p], GRP_A, NGRP)
            ob.start()

    start_in(0, 0)
    for it in range(NITER):
        p = it & 1
        q = 1 - p
        if it >= 1:
            wait_out(it - 1, q)
        if it + 1 < NITER:
            start_in(it + 1, q)
        process(it, p)
    wait_out(NITER - 1, (NITER - 1) & 1)


def kernel(x, batch):
    batch = batch.astype(jnp.int32)
    mesh = plsc.VectorSubcoreMesh(core_axis_name="c", subcore_axis_name="s")
    out = pl.kernel(
        _sc_body,
        out_type=jax.ShapeDtypeStruct((N_NODES, D_FEAT), jnp.float32),
        mesh=mesh,
        scratch_types=[
            pltpu.VMEM((CHUNK, D_FEAT), jnp.float32),
            pltpu.VMEM((CHUNK, D_FEAT), jnp.float32),
            pltpu.VMEM((CHUNK,), jnp.int32),
            pltpu.VMEM((CHUNK,), jnp.int32),
            pltpu.VMEM((G_PAD,), jnp.float32),
        ] + [pltpu.SemaphoreType.DMA] * 8,
    )(x, batch)
    return out


# speedup vs baseline: 1.8922x; 1.8922x over previous
import jax
import jax.numpy as jnp
from jax import lax
from jax.experimental import pallas as pl
from jax.experimental.pallas import tpu as pltpu
from jax.experimental.pallas import tpu_sc as plsc

N_NODES = 100000
D_FEAT = 128
N_GRAPHS = 1024

NC = 2
NS = 16
NW = NC * NS
LANES = 16

CHUNK = 400
HALF_A = 192
HALF_DMA = 200
NCHUNKS = N_NODES // CHUNK
NITER = (NCHUNKS + NW - 1) // NW
NGRP = CHUNK // LANES
GRP_A = HALF_A // LANES
G_PAD = N_GRAPHS + LANES


def _sc_body(x_hbm, batch_hbm, out_hbm,
             xb0, xb1, bb0, bb1, drop,
             isa0, isa1, isb0, isb1, osa0, osa1, osb0, osb1):
    wid = lax.axis_index("s") * NC + lax.axis_index("c")
    xbs, bbs = (xb0, xb1), (bb0, bb1)
    isemA, isemB = (isa0, isa1), (isb0, isb1)
    osemA, osemB = (osa0, osa1), (osb0, osb1)

    ones16 = jnp.full((LANES,), 1.0, jnp.float32)

    def init_body(g, carry):
        drop[pl.ds(g * LANES, LANES)] = ones16
        return carry

    lax.fori_loop(0, G_PAD // LANES, init_body, 0)

    def guard(it):
        return wid + it * NW < NCHUNKS

    def in_copies(it, p):
        row0 = (wid + it * NW) * CHUNK
        return (
            pltpu.make_async_copy(x_hbm.at[pl.ds(row0, HALF_DMA), :],
                                  xbs[p].at[pl.ds(0, HALF_DMA), :], isemA[p]),
            pltpu.make_async_copy(batch_hbm.at[pl.ds(row0, CHUNK)], bbs[p], isemA[p]),
            pltpu.make_async_copy(x_hbm.at[pl.ds(row0 + HALF_DMA, HALF_DMA), :],
                                  xbs[p].at[pl.ds(HALF_DMA, HALF_DMA), :], isemB[p]),
        )

    def out_copies(it, p):
        row0 = (wid + it * NW) * CHUNK
        return (
            pltpu.make_async_copy(xbs[p].at[pl.ds(0, HALF_A), :],
                                  out_hbm.at[pl.ds(row0, HALF_A), :], osemA[p]),
            pltpu.make_async_copy(xbs[p].at[pl.ds(HALF_A, CHUNK - HALF_A), :],
                                  out_hbm.at[pl.ds(row0 + HALF_A, CHUNK - HALF_A), :],
                                  osemB[p]),
        )

    def start_in(it, p):
        @pl.when(guard(it))
        def _():
            ca, cb, cc = in_copies(it, p)
            ca.start()
            cb.start()
            cc.start()

    def wait_out(it, p):
        @pl.when(guard(it))
        def _():
            oa, ob = out_copies(it, p)
            oa.wait()
            ob.wait()

    def compute_groups(xbuf, bbuf, g_lo, g_hi):
        @plsc.parallel_loop(g_lo, g_hi)
        def _(g):
            bvec = bbuf[pl.ds(g * LANES, LANES)]
            b_first = bvec[0]
            b_last = bvec[LANES - 1]

            @pl.when(b_first == b_last)
            def _():
                dval = drop[pl.ds(b_first, LANES)][0]
                for j in range(LANES):
                    r = g * LANES + j
                    for k in range(D_FEAT // LANES):
                        xv = xbuf[r, pl.ds(k * LANES, LANES)]
                        xbuf[r, pl.ds(k * LANES, LANES)] = xv * dval

            @pl.when(b_first != b_last)
            def _():
                for j in range(LANES):
                    b = bvec[j]
                    dval = drop[pl.ds(b, LANES)][0]
                    r = g * LANES + j
                    for k in range(D_FEAT // LANES):
                        xv = xbuf[r, pl.ds(k * LANES, LANES)]
                        xbuf[r, pl.ds(k * LANES, LANES)] = xv * dval

    def process(it, p):
        @pl.when(guard(it))
        def _():
            ca, cb, _cc = in_copies(it, p)
            oa, ob = out_copies(it, p)
            ca.wait()
            cb.wait()
            compute_groups(xbs[p], bbs[p], 0, GRP_A)
            oa.start()
            _ca, _cb, cc = in_copies(it, p)
            cc.wait()
            compute_groups(xbs[p], bbs[p], GRP_A, NGRP)
            ob.start()

    start_in(0, 0)

    def pair_body(t, carry):
        it0 = 2 * t
        it1 = it0 + 1

        @pl.when(t >= 1)
        def _():
            wait_out(it0 - 1, 1)
        start_in(it0 + 1, 1)
        process(it0, 0)
        wait_out(it1 - 1, 0)

        @pl.when(t < NITER // 2 - 1)
        def _():
            start_in(it1 + 1, 0)
        process(it1, 1)
        return carry

    lax.fori_loop(0, NITER // 2, pair_body, 0)
    wait_out(NITER - 1, (NITER - 1) & 1)


def kernel(x, batch):
    batch = batch.astype(jnp.int32)
    mesh = plsc.VectorSubcoreMesh(core_axis_name="c", subcore_axis_name="s")
    out = pl.kernel(
        _sc_body,
        out_type=jax.ShapeDtypeStruct((N_NODES, D_FEAT), jnp.float32),
        mesh=mesh,
        scratch_types=[
            pltpu.VMEM((CHUNK, D_FEAT), jnp.float32),
            pltpu.VMEM((CHUNK, D_FEAT), jnp.float32),
            pltpu.VMEM((CHUNK,), jnp.int32),
            pltpu.VMEM((CHUNK,), jnp.int32),
            pltpu.VMEM((G_PAD,), jnp.float32),
        ] + [pltpu.SemaphoreType.DMA] * 8,
    )(x, batch)
    return out
